# baseline (speedup 1.0000x reference)
import functools

import jax
import jax.numpy as jnp
from jax import lax
from jax.experimental import pallas as pl
from jax.experimental.pallas import tpu as pltpu
from jax.experimental.pallas import tpu_sc as plsc

_V = 100000
_D = 512
_N = 64
_CTX = 2
_G = _D // 4
_BEAM = 8
_BV = 4000
_NB = _V // _BV
_NSLOT = 15
_NEG = -1e30
_IBIG = 2 ** 30


def _sc_gather(emb, ids):
    mesh = plsc.VectorSubcoreMesh(core_axis_name="c", subcore_axis_name="s")

    @functools.partial(
        pl.kernel, mesh=mesh,
        out_type=jax.ShapeDtypeStruct((2 * _N, _D), jnp.float32),
        scratch_types=[
            pltpu.VMEM((8,), jnp.int32),
            pltpu.VMEM((8, _D), jnp.float32),
            pltpu.SemaphoreType.DMA,
        ],
    )
    def k(emb_hbm, idx_hbm, out_hbm, idx_v, rows_v, sem):
        wid = lax.axis_index("s") * 2 + lax.axis_index("c")

        @pl.when(wid < 16)
        def _():
            base = wid * 8
            pltpu.sync_copy(idx_hbm.at[pl.ds(base, 8)], idx_v)
            pltpu.async_copy(emb_hbm.at[idx_v], rows_v, sem).wait()
            pltpu.sync_copy(rows_v, out_hbm.at[pl.ds(base, 8)])

    return k(emb, ids)


def _body(e01_ref, enc_ref, hlp_ref, m0_ref, m1_ref, wp_ref, bp_ref,
          wj_ref, bj_ref,
          outv_ref, outp_ref, outh_ref, outt_ref,
          joint_s, m_s, s_s, candv_s, candi_s, t8_s, logits_s):
    i = pl.program_id(0)
    nt = (((1,), (1,)), ((), ()))

    @pl.when(i == 0)
    def _init():
        e0 = e01_ref[0:_N, :]
        e1 = e01_ref[_N:2 * _N, :]
        nn = (((1,), (0,)), ((), ()))
        d = lax.dot_general(e0, m0_ref[...], nn,
                            preferred_element_type=jnp.float32)
        d += lax.dot_general(e1, m1_ref[...], nn,
                             preferred_element_type=jnp.float32)
        d = jnp.maximum(d, 0.0)
        p = lax.dot_general(d, wp_ref[...], nt,
                            preferred_element_type=jnp.float32)
        joint_s[...] = jnp.tanh(enc_ref[...] + p + bp_ref[...])
        m_s[...] = jnp.full((_N, 128), _NEG, jnp.float32)
        s_s[...] = jnp.zeros((_N, 128), jnp.float32)
        candv_s[...] = jnp.full((_N, 128), _NEG, jnp.float32)
        candi_s[...] = jnp.zeros((_N, 128), jnp.int32)
        t8_s[...] = jnp.full((_N, 128), _NEG, jnp.float32)

    slot = lax.rem(i, 2)

    @pl.when(i < _NB)
    def _compute():
        lg = lax.dot_general(joint_s[...], wj_ref[...], nt,
                             preferred_element_type=jnp.float32)
        lg += lax.dot_general(jnp.ones((_N, 1), jnp.float32), bj_ref[0], nt,
                              preferred_element_type=jnp.float32)
        logits_s[slot] = lg

    @pl.when(i > 0)
    def _consume():
        b = i - 1
        v = logits_s[1 - slot]

        bm = jnp.max(v, axis=1, keepdims=True)
        m_old = m_s[...][:, 0:1]
        m_new = jnp.maximum(m_old, bm)
        sumexp = jnp.sum(jnp.exp(v - m_new), axis=1, keepdims=True)
        s_new = s_s[...][:, 0:1] * jnp.exp(m_old - m_new) + sumexp
        m_s[...] = jnp.broadcast_to(m_new, (_N, 128))
        s_s[...] = jnp.broadcast_to(s_new, (_N, 128))

        jj = lax.rem(b, _NSLOT)
        lane = lax.broadcasted_iota(jnp.int32, (_N, 128), 1)
        colid = lax.broadcasted_iota(jnp.int32, (_N, _BV), 1)
        t8_old = t8_s[...][:, 0:1]
        cnt = jnp.sum((v > t8_old).astype(jnp.int32), axis=1, keepdims=True)
        n_it = jnp.minimum(jnp.max(cnt), _BEAM)

        def _extract(k, carry):
            v, candv, candi, _ = carry
            mk = jnp.max(v, axis=1, keepdims=True)
            c = jnp.min(jnp.where(v == mk, colid, _IBIG),
                        axis=1, keepdims=True)
            lsel = lane == (8 + 8 * jj + k)
            candv = jnp.where(lsel, mk, candv)
            candi = jnp.where(lsel, b * _BV + c, candi)
            v = jnp.where(colid == c, _NEG, v)
            return v, candv, candi, mk

        _, candv, candi, last_mk = lax.fori_loop(
            0, n_it, _extract,
            (v, candv_s[...], candi_s[...], bm))
        t8_s[...] = jnp.broadcast_to(
            jnp.where(n_it >= _BEAM,
                      jnp.maximum(t8_old, last_mk), t8_old), (_N, 128))

        @pl.when(jj == _NSLOT - 1)
        def _merge():
            w = candv
            nv = jnp.full((_N, 128), _NEG, jnp.float32)
            ni = jnp.zeros((_N, 128), jnp.int32)
            for k in range(_BEAM):
                mk = jnp.max(w, axis=1, keepdims=True)
                c = jnp.min(jnp.where(w == mk, lane, _IBIG),
                            axis=1, keepdims=True)
                ci = jnp.min(jnp.where(lane == c, candi, _IBIG),
                             axis=1, keepdims=True)
                nv = jnp.where(lane == k, mk, nv)
                ni = jnp.where(lane == k, ci, ni)
                w = jnp.where(lane == c, _NEG, w)
            candv_s[...] = nv
            candi_s[...] = ni

        @pl.when(jj != _NSLOT - 1)
        def _store():
            candv_s[...] = candv
            candi_s[...] = candi

    @pl.when(i == _NB)
    def _final():
        lane2 = lax.broadcasted_iota(jnp.int32, (_N, 128), 1)
        rowi = lax.broadcasted_iota(jnp.int32, (_N, 128), 0)
        m = m_s[...][:, 0:1]
        s = s_s[...][:, 0:1]
        lse = m + jnp.log(s)
        hlp_b = hlp_ref[...]
        adj = candv_s[...] + hlp_b - lse
        ci = candi_s[...]
        flat = rowi * _V + ci
        laneo = lax.broadcasted_iota(jnp.int32, (1, 128), 1)
        ov = jnp.full((1, 128), 0.0, jnp.float32)
        op = jnp.full((1, 128), 0.0, jnp.float32)
        oh = jnp.zeros((1, 128), jnp.int32)
        ot = jnp.zeros((1, 128), jnp.int32)
        for k in range(_BEAM):
            g = jnp.max(adj)
            hit = adj == g
            f = jnp.min(jnp.where(hit, flat, _IBIG))
            sel = hit & (flat == f)
            hlp_at = jnp.min(jnp.where(sel, hlp_b, jnp.float32(_IBIG)))
            tok = jnp.min(jnp.where(sel, ci, _IBIG))
            hyp = jnp.min(jnp.where(sel, rowi, _IBIG))
            ov = jnp.where(laneo == k, g, ov)
            op = jnp.where(laneo == k, jnp.exp(g - hlp_at), op)
            oh = jnp.where(laneo == k, hyp, oh)
            ot = jnp.where(laneo == k, tok, ot)
            adj = jnp.where(sel, _NEG, adj)
        outv_ref[...] = ov
        outp_ref[...] = op
        outh_ref[...] = oh
        outt_ref[...] = ot


def kernel(decoder_input, encoder_out, hyps_log_prob, emb, conv_w, Wp, bp,
           Wj, bj):
    f32 = jnp.float32
    cw = conv_w.reshape(_G, 4, 4, _CTX)
    eye = jnp.eye(_G, dtype=f32)
    m0 = jnp.einsum('goi,gh->giho', cw[..., 0], eye).reshape(_D, _D)
    m1 = jnp.einsum('goi,gh->giho', cw[..., 1], eye).reshape(_D, _D)

    ids = decoder_input.T.reshape(-1)
    e01 = _sc_gather(emb, ids)

    hlp_b = jnp.broadcast_to(hyps_log_prob, (_N, 128))
    bp2 = bp.reshape(1, _D)
    bj2 = bj.reshape(_NB, _BV, 1)

    last = _NB - 1
    outs = pl.pallas_call(
        _body,
        grid=(_NB + 1,),
        in_specs=[
            pl.BlockSpec((2 * _N, _D), lambda i: (0, 0)),
            pl.BlockSpec((_N, _D), lambda i: (0, 0)),
            pl.BlockSpec((_N, 128), lambda i: (0, 0)),
            pl.BlockSpec((_D, _D), lambda i: (0, 0)),
            pl.BlockSpec((_D, _D), lambda i: (0, 0)),
            pl.BlockSpec((_D, _D), lambda i: (0, 0)),
            pl.BlockSpec((1, _D), lambda i: (0, 0)),
            pl.BlockSpec((_BV, _D), lambda i: (jnp.minimum(i, last), 0)),
            pl.BlockSpec((1, _BV, 1), lambda i: (jnp.minimum(i, last), 0, 0)),
        ],
        out_specs=[pl.BlockSpec((1, 128), lambda i: (0, 0))] * 4,
        out_shape=[
            jax.ShapeDtypeStruct((1, 128), f32),
            jax.ShapeDtypeStruct((1, 128), f32),
            jax.ShapeDtypeStruct((1, 128), jnp.int32),
            jax.ShapeDtypeStruct((1, 128), jnp.int32),
        ],
        scratch_shapes=[
            pltpu.VMEM((_N, _D), f32),
            pltpu.VMEM((_N, 128), f32),
            pltpu.VMEM((_N, 128), f32),
            pltpu.VMEM((_N, 128), f32),
            pltpu.VMEM((_N, 128), jnp.int32),
            pltpu.VMEM((_N, 128), f32),
            pltpu.VMEM((2, _N, _BV), f32),
        ],
    )(e01, encoder_out, hlp_b, m0, m1, Wp, bp2, Wj, bj2)

    ov, op, oh, ot = outs
    return (ov[0, :_BEAM], op[0, :_BEAM], oh[0, :_BEAM], ot[0, :_BEAM])


# speedup vs baseline: 1.1880x; 1.1880x over previous
import functools

import jax
import jax.numpy as jnp
from jax import lax
from jax.experimental import pallas as pl
from jax.experimental.pallas import tpu as pltpu
from jax.experimental.pallas import tpu_sc as plsc

_V = 100000
_D = 512
_N = 64
_CTX = 2
_G = _D // 4
_BEAM = 8
_BV = 5000
_NB = _V // _BV
_NSLOT = 15
_NEG = -1e30
_IBIG = 2 ** 30


def _sc_gather(emb, ids):
    mesh = plsc.VectorSubcoreMesh(core_axis_name="c", subcore_axis_name="s")

    @functools.partial(
        pl.kernel, mesh=mesh,
        out_type=jax.ShapeDtypeStruct((2 * _N, _D), jnp.float32),
        scratch_types=[
            pltpu.VMEM((8,), jnp.int32),
            pltpu.VMEM((8, _D), jnp.float32),
            pltpu.SemaphoreType.DMA,
        ],
    )
    def k(emb_hbm, idx_hbm, out_hbm, idx_v, rows_v, sem):
        wid = lax.axis_index("s") * 2 + lax.axis_index("c")

        @pl.when(wid < 16)
        def _():
            base = wid * 8
            pltpu.sync_copy(idx_hbm.at[pl.ds(base, 8)], idx_v)
            pltpu.async_copy(emb_hbm.at[idx_v], rows_v, sem).wait()
            pltpu.sync_copy(rows_v, out_hbm.at[pl.ds(base, 8)])

    return k(emb, ids)


def _body(e01_ref, enc_ref, hlp_ref, m0_ref, m1_ref, wp_ref, bp_ref,
          wj_ref, bj_ref,
          outv_ref, outp_ref, outh_ref, outt_ref,
          joint_s, m_s, s_s, candv_s, candi_s, logits_s):
    i = pl.program_id(0)
    nt = (((1,), (1,)), ((), ()))

    @pl.when(i == 0)
    def _init():
        e0 = e01_ref[0:_N, :]
        e1 = e01_ref[_N:2 * _N, :]
        nn = (((1,), (0,)), ((), ()))
        d = lax.dot_general(e0, m0_ref[...], nn,
                            preferred_element_type=jnp.float32)
        d += lax.dot_general(e1, m1_ref[...], nn,
                             preferred_element_type=jnp.float32)
        d = jnp.maximum(d, 0.0)
        p = lax.dot_general(d, wp_ref[...], nt,
                            preferred_element_type=jnp.float32)
        joint_s[...] = jnp.tanh(enc_ref[...] + p + bp_ref[...])
        m_s[...] = jnp.full((_N, 128), _NEG, jnp.float32)
        s_s[...] = jnp.zeros((_N, 128), jnp.float32)
        candv_s[...] = jnp.full((_N, 128), _NEG, jnp.float32)
        candi_s[...] = jnp.zeros((_N, 128), jnp.int32)

    slot = lax.rem(i, 2)

    @pl.when(i < _NB)
    def _compute():
        lg = lax.dot_general(joint_s[...], wj_ref[...], nt,
                             preferred_element_type=jnp.float32)
        lg += lax.dot_general(jnp.ones((_N, 1), jnp.float32), bj_ref[0], nt,
                              preferred_element_type=jnp.float32)
        logits_s[slot] = lg

    @pl.when(i > 0)
    def _consume():
        b = i - 1
        v = logits_s[1 - slot]

        bm = jnp.max(v, axis=1, keepdims=True)
        m_old = m_s[...][:, 0:1]
        m_new = jnp.maximum(m_old, bm)
        sumexp = jnp.sum(jnp.exp(v - m_new), axis=1, keepdims=True)
        s_new = s_s[...][:, 0:1] * jnp.exp(m_old - m_new) + sumexp
        m_s[...] = jnp.broadcast_to(m_new, (_N, 128))
        s_s[...] = jnp.broadcast_to(s_new, (_N, 128))

        jj = lax.rem(b, _NSLOT)
        lane = lax.broadcasted_iota(jnp.int32, (_N, 128), 1)
        colid = lax.broadcasted_iota(jnp.int32, (_N, _BV), 1)
        candv = candv_s[...]
        candi = candi_s[...]
        for k in range(_BEAM):
            mk = bm if k == 0 else jnp.max(v, axis=1, keepdims=True)
            c = jnp.min(jnp.where(v == mk, colid, _IBIG),
                        axis=1, keepdims=True)
            lsel = lane == (8 + 8 * jj + k)
            candv = jnp.where(lsel, mk, candv)
            candi = jnp.where(lsel, b * _BV + c, candi)
            v = jnp.where(colid == c, _NEG, v)

        @pl.when(jj == _NSLOT - 1)
        def _merge():
            w = candv
            nv = jnp.full((_N, 128), _NEG, jnp.float32)
            ni = jnp.zeros((_N, 128), jnp.int32)
            for k in range(_BEAM):
                mk = jnp.max(w, axis=1, keepdims=True)
                c = jnp.min(jnp.where(w == mk, lane, _IBIG),
                            axis=1, keepdims=True)
                ci = jnp.min(jnp.where(lane == c, candi, _IBIG),
                             axis=1, keepdims=True)
                nv = jnp.where(lane == k, mk, nv)
                ni = jnp.where(lane == k, ci, ni)
                w = jnp.where(lane == c, _NEG, w)
            candv_s[...] = nv
            candi_s[...] = ni

        @pl.when(jj != _NSLOT - 1)
        def _store():
            candv_s[...] = candv
            candi_s[...] = candi

    @pl.when(i == _NB)
    def _final():
        lane2 = lax.broadcasted_iota(jnp.int32, (_N, 128), 1)
        rowi = lax.broadcasted_iota(jnp.int32, (_N, 128), 0)
        m = m_s[...][:, 0:1]
        s = s_s[...][:, 0:1]
        lse = m + jnp.log(s)
        hlp_b = hlp_ref[...]
        adj = candv_s[...] + hlp_b - lse
        ci = candi_s[...]
        flat = rowi * _V + ci
        laneo = lax.broadcasted_iota(jnp.int32, (1, 128), 1)
        ov = jnp.full((1, 128), 0.0, jnp.float32)
        op = jnp.full((1, 128), 0.0, jnp.float32)
        oh = jnp.zeros((1, 128), jnp.int32)
        ot = jnp.zeros((1, 128), jnp.int32)
        for k in range(_BEAM):
            g = jnp.max(adj)
            hit = adj == g
            f = jnp.min(jnp.where(hit, flat, _IBIG))
            sel = hit & (flat == f)
            hlp_at = jnp.min(jnp.where(sel, hlp_b, jnp.float32(_IBIG)))
            tok = jnp.min(jnp.where(sel, ci, _IBIG))
            hyp = jnp.min(jnp.where(sel, rowi, _IBIG))
            ov = jnp.where(laneo == k, g, ov)
            op = jnp.where(laneo == k, jnp.exp(g - hlp_at), op)
            oh = jnp.where(laneo == k, hyp, oh)
            ot = jnp.where(laneo == k, tok, ot)
            adj = jnp.where(sel, _NEG, adj)
        outv_ref[...] = ov
        outp_ref[...] = op
        outh_ref[...] = oh
        outt_ref[...] = ot


def kernel(decoder_input, encoder_out, hyps_log_prob, emb, conv_w, Wp, bp,
           Wj, bj):
    f32 = jnp.float32
    cw = conv_w.reshape(_G, 4, 4, _CTX)
    eye = jnp.eye(_G, dtype=f32)
    m0 = jnp.einsum('goi,gh->giho', cw[..., 0], eye).reshape(_D, _D)
    m1 = jnp.einsum('goi,gh->giho', cw[..., 1], eye).reshape(_D, _D)

    ids = decoder_input.T.reshape(-1)
    e01 = _sc_gather(emb, ids)

    hlp_b = jnp.broadcast_to(hyps_log_prob, (_N, 128))
    bp2 = bp.reshape(1, _D)
    bj2 = bj.reshape(_NB, _BV, 1)

    last = _NB - 1
    outs = pl.pallas_call(
        _body,
        grid=(_NB + 1,),
        in_specs=[
            pl.BlockSpec((2 * _N, _D), lambda i: (0, 0)),
            pl.BlockSpec((_N, _D), lambda i: (0, 0)),
            pl.BlockSpec((_N, 128), lambda i: (0, 0)),
            pl.BlockSpec((_D, _D), lambda i: (0, 0)),
            pl.BlockSpec((_D, _D), lambda i: (0, 0)),
            pl.BlockSpec((_D, _D), lambda i: (0, 0)),
            pl.BlockSpec((1, _D), lambda i: (0, 0)),
            pl.BlockSpec((_BV, _D), lambda i: (jnp.minimum(i, last), 0)),
            pl.BlockSpec((1, _BV, 1), lambda i: (jnp.minimum(i, last), 0, 0)),
        ],
        out_specs=[pl.BlockSpec((1, 128), lambda i: (0, 0))] * 4,
        out_shape=[
            jax.ShapeDtypeStruct((1, 128), f32),
            jax.ShapeDtypeStruct((1, 128), f32),
            jax.ShapeDtypeStruct((1, 128), jnp.int32),
            jax.ShapeDtypeStruct((1, 128), jnp.int32),
        ],
        scratch_shapes=[
            pltpu.VMEM((_N, _D), f32),
            pltpu.VMEM((_N, 128), f32),
            pltpu.VMEM((_N, 128), f32),
            pltpu.VMEM((_N, 128), f32),
            pltpu.VMEM((_N, 128), jnp.int32),
            pltpu.VMEM((2, _N, _BV), f32),
        ],
    )(e01, encoder_out, hlp_b, m0, m1, Wp, bp2, Wj, bj2)

    ov, op, oh, ot = outs
    return (ov[0, :_BEAM], op[0, :_BEAM], oh[0, :_BEAM], ot[0, :_BEAM])


# speedup vs baseline: 1.2733x; 1.0718x over previous
import functools

import jax
import jax.numpy as jnp
from jax import lax
from jax.experimental import pallas as pl
from jax.experimental.pallas import tpu as pltpu
from jax.experimental.pallas import tpu_sc as plsc

_V = 100000
_D = 512
_N = 64
_CTX = 2
_G = _D // 4
_BEAM = 8
_BV = 4992
_NB = -(-_V // _BV)
_CH = _BV // 128
_NEG = -1e30
_IBIG = 2 ** 30


def _sc_gather(emb, ids):
    mesh = plsc.VectorSubcoreMesh(core_axis_name="c", subcore_axis_name="s")

    @functools.partial(
        pl.kernel, mesh=mesh,
        out_type=jax.ShapeDtypeStruct((2 * _N, _D), jnp.float32),
        scratch_types=[
            pltpu.VMEM((8,), jnp.int32),
            pltpu.VMEM((8, _D), jnp.float32),
            pltpu.SemaphoreType.DMA,
        ],
    )
    def k(emb_hbm, idx_hbm, out_hbm, idx_v, rows_v, sem):
        wid = lax.axis_index("s") * 2 + lax.axis_index("c")

        @pl.when(wid < 16)
        def _():
            base = wid * 8
            pltpu.sync_copy(idx_hbm.at[pl.ds(base, 8)], idx_v)
            pltpu.async_copy(emb_hbm.at[idx_v], rows_v, sem).wait()
            pltpu.sync_copy(rows_v, out_hbm.at[pl.ds(base, 8)])

    return k(emb, ids)


def _body(e01_ref, enc_ref, hlp_ref, m0_ref, m1_ref, wp_ref, bp_ref,
          wj_ref, bj_ref,
          outv_ref, outp_ref, outh_ref, outt_ref,
          joint_s, m_s, s_s, mv_s, mi_s, logits_s):
    i = pl.program_id(0)
    nt = (((1,), (1,)), ((), ()))

    @pl.when(i == 0)
    def _init():
        e0 = e01_ref[0:_N, :]
        e1 = e01_ref[_N:2 * _N, :]
        nn = (((1,), (0,)), ((), ()))
        d = lax.dot_general(e0, m0_ref[...], nn,
                            preferred_element_type=jnp.float32)
        d += lax.dot_general(e1, m1_ref[...], nn,
                             preferred_element_type=jnp.float32)
        d = jnp.maximum(d, 0.0)
        p = lax.dot_general(d, wp_ref[...], nt,
                            preferred_element_type=jnp.float32)
        joint_s[...] = jnp.tanh(enc_ref[...] + p + bp_ref[...])
        m_s[...] = jnp.full((_N, 128), _NEG, jnp.float32)
        s_s[...] = jnp.zeros((_N, 128), jnp.float32)
        mv_s[...] = jnp.full((_BEAM, _N, 128), _NEG, jnp.float32)
        mi_s[...] = jnp.zeros((_BEAM, _N, 128), jnp.int32)

    slot = lax.rem(i, 2)

    @pl.when(i < _NB)
    def _compute():
        lg = lax.dot_general(joint_s[...], wj_ref[...], nt,
                             preferred_element_type=jnp.float32)
        lg += lax.dot_general(jnp.ones((_N, 1), jnp.float32), bj_ref[0], nt,
                              preferred_element_type=jnp.float32)
        colid = lax.broadcasted_iota(jnp.int32, (_N, _BV), 1)
        lim = jnp.where(i == _NB - 1, _V - (_NB - 1) * _BV, _BV)
        logits_s[slot] = jnp.where(colid < lim, lg, _NEG)

    @pl.when(i > 0)
    def _consume():
        b = i - 1
        v = logits_s[1 - slot]

        bm = jnp.max(v, axis=1, keepdims=True)
        m_old = m_s[...][:, 0:1]
        m_new = jnp.maximum(m_old, bm)
        sumexp = jnp.sum(jnp.exp(v - m_new), axis=1, keepdims=True)
        s_new = s_s[...][:, 0:1] * jnp.exp(m_old - m_new) + sumexp
        m_s[...] = jnp.broadcast_to(m_new, (_N, 128))
        s_s[...] = jnp.broadcast_to(s_new, (_N, 128))

        ms = [mv_s[j] for j in range(_BEAM)]
        js = [mi_s[j] for j in range(_BEAM)]
        for t in range(_CH):
            tv = logits_s[1 - slot, :, 128 * t:128 * (t + 1)]
            ti = jnp.full((_N, 128), b * _BV + 128 * t, jnp.int32)
            for j in range(_BEAM):
                hi = jnp.maximum(tv, ms[j])
                lo = jnp.minimum(tv, ms[j])
                cmp = tv > ms[j]
                hi_i = jnp.where(cmp, ti, js[j])
                lo_i = jnp.where(cmp, js[j], ti)
                ms[j], tv, js[j], ti = hi, lo, hi_i, lo_i
        for j in range(_BEAM):
            mv_s[j] = ms[j]
            mi_s[j] = js[j]

    @pl.when(i == _NB)
    def _final():
        lane3 = lax.broadcasted_iota(jnp.int32, (_BEAM, _N, 128), 2)
        rowi = lax.broadcasted_iota(jnp.int32, (_BEAM, _N, 128), 1)
        m = m_s[...][:, 0:1]
        s = s_s[...][:, 0:1]
        lse = m + jnp.log(s)
        hlp_b = hlp_ref[...]
        adj = mv_s[...] + (hlp_b - lse)[None]
        hlp3 = jnp.broadcast_to(hlp_b[None], (_BEAM, _N, 128))
        ci = mi_s[...] + lane3
        flat = rowi * _V + ci
        laneo = lax.broadcasted_iota(jnp.int32, (1, 128), 1)
        ov = jnp.full((1, 128), 0.0, jnp.float32)
        op = jnp.full((1, 128), 0.0, jnp.float32)
        oh = jnp.zeros((1, 128), jnp.int32)
        ot = jnp.zeros((1, 128), jnp.int32)
        for k in range(_BEAM):
            g = jnp.max(adj)
            hit = adj == g
            f = jnp.min(jnp.where(hit, flat, _IBIG))
            sel = hit & (flat == f)
            hlp_at = jnp.min(jnp.where(sel, hlp3, jnp.float32(_IBIG)))
            tok = jnp.min(jnp.where(sel, ci, _IBIG))
            hyp = jnp.min(jnp.where(sel, rowi, _IBIG))
            ov = jnp.where(laneo == k, g, ov)
            op = jnp.where(laneo == k, jnp.exp(g - hlp_at), op)
            oh = jnp.where(laneo == k, hyp, oh)
            ot = jnp.where(laneo == k, tok, ot)
            adj = jnp.where(sel, _NEG, adj)
        outv_ref[...] = ov
        outp_ref[...] = op
        outh_ref[...] = oh
        outt_ref[...] = ot


def kernel(decoder_input, encoder_out, hyps_log_prob, emb, conv_w, Wp, bp,
           Wj, bj):
    f32 = jnp.float32
    cw = conv_w.reshape(_G, 4, 4, _CTX)
    eye = jnp.eye(_G, dtype=f32)
    m0 = jnp.einsum('goi,gh->giho', cw[..., 0], eye).reshape(_D, _D)
    m1 = jnp.einsum('goi,gh->giho', cw[..., 1], eye).reshape(_D, _D)

    ids = decoder_input.T.reshape(-1)
    e01 = _sc_gather(emb, ids)

    hlp_b = jnp.broadcast_to(hyps_log_prob, (_N, 128))
    bp2 = bp.reshape(1, _D)
    bj2 = jnp.pad(bj, (0, _NB * _BV - _V)).reshape(_NB, _BV, 1)

    last = _NB - 1
    outs = pl.pallas_call(
        _body,
        grid=(_NB + 1,),
        in_specs=[
            pl.BlockSpec((2 * _N, _D), lambda i: (0, 0)),
            pl.BlockSpec((_N, _D), lambda i: (0, 0)),
            pl.BlockSpec((_N, 128), lambda i: (0, 0)),
            pl.BlockSpec((_D, _D), lambda i: (0, 0)),
            pl.BlockSpec((_D, _D), lambda i: (0, 0)),
            pl.BlockSpec((_D, _D), lambda i: (0, 0)),
            pl.BlockSpec((1, _D), lambda i: (0, 0)),
            pl.BlockSpec((_BV, _D), lambda i: (jnp.minimum(i, last), 0)),
            pl.BlockSpec((1, _BV, 1), lambda i: (jnp.minimum(i, last), 0, 0)),
        ],
        out_specs=[pl.BlockSpec((1, 128), lambda i: (0, 0))] * 4,
        out_shape=[
            jax.ShapeDtypeStruct((1, 128), f32),
            jax.ShapeDtypeStruct((1, 128), f32),
            jax.ShapeDtypeStruct((1, 128), jnp.int32),
            jax.ShapeDtypeStruct((1, 128), jnp.int32),
        ],
        scratch_shapes=[
            pltpu.VMEM((_N, _D), f32),
            pltpu.VMEM((_N, 128), f32),
            pltpu.VMEM((_N, 128), f32),
            pltpu.VMEM((_BEAM, _N, 128), f32),
            pltpu.VMEM((_BEAM, _N, 128), jnp.int32),
            pltpu.VMEM((2, _N, _BV), f32),
        ],
    )(e01, encoder_out, hlp_b, m0, m1, Wp, bp2, Wj, bj2)

    ov, op, oh, ot = outs
    return (ov[0, :_BEAM], op[0, :_BEAM], oh[0, :_BEAM], ot[0, :_BEAM])


# speedup vs baseline: 1.2743x; 1.0008x over previous
import functools

import jax
import jax.numpy as jnp
from jax import lax
from jax.experimental import pallas as pl
from jax.experimental.pallas import tpu as pltpu
from jax.experimental.pallas import tpu_sc as plsc

_V = 100000
_D = 512
_N = 64
_CTX = 2
_G = _D // 4
_BEAM = 8
_BV = 4992
_NB = -(-_V // _BV)
_CH = _BV // 128
_NEG = -1e30
_IBIG = 2 ** 30


def _sc_gather(emb, ids):
    mesh = plsc.VectorSubcoreMesh(core_axis_name="c", subcore_axis_name="s")

    @functools.partial(
        pl.kernel, mesh=mesh,
        out_type=jax.ShapeDtypeStruct((2 * _N, _D), jnp.float32),
        scratch_types=[
            pltpu.VMEM((8,), jnp.int32),
            pltpu.VMEM((8, _D), jnp.float32),
            pltpu.SemaphoreType.DMA,
        ],
    )
    def k(emb_hbm, idx_hbm, out_hbm, idx_v, rows_v, sem):
        wid = lax.axis_index("s") * 2 + lax.axis_index("c")

        @pl.when(wid < 16)
        def _():
            base = wid * 8
            pltpu.sync_copy(idx_hbm.at[pl.ds(base, 8)], idx_v)
            pltpu.async_copy(emb_hbm.at[idx_v], rows_v, sem).wait()
            pltpu.sync_copy(rows_v, out_hbm.at[pl.ds(base, 8)])

    return k(emb, ids)


def _body(e01_ref, enc_ref, hlp_ref, m0_ref, m1_ref, wp_ref, bp_ref,
          wj_ref, bj_ref,
          outv_ref, outp_ref, outh_ref, outt_ref,
          joint_s, m_s, s_s, mv_s, mi_s, logits_s):
    i = pl.program_id(0)
    nt = (((1,), (1,)), ((), ()))

    @pl.when(i == 0)
    def _init():
        e0 = e01_ref[0:_N, :]
        e1 = e01_ref[_N:2 * _N, :]
        nn = (((1,), (0,)), ((), ()))
        d = lax.dot_general(e0, m0_ref[...], nn,
                            preferred_element_type=jnp.float32)
        d += lax.dot_general(e1, m1_ref[...], nn,
                             preferred_element_type=jnp.float32)
        d = jnp.maximum(d, 0.0)
        p = lax.dot_general(d, wp_ref[...], nt,
                            preferred_element_type=jnp.float32)
        joint_s[...] = jnp.tanh(enc_ref[...] + p + bp_ref[...])
        m_s[...] = jnp.full((_N, 128), _NEG, jnp.float32)
        s_s[...] = jnp.zeros((_N, 128), jnp.float32)
        mv_s[...] = jnp.full((_BEAM, _N, 128), _NEG, jnp.float32)
        mi_s[...] = jnp.zeros((_BEAM, _N, 128), jnp.int32)

    slot = lax.rem(i, 2)

    @pl.when(i < _NB)
    def _compute():
        lg = lax.dot_general(joint_s[...], wj_ref[...], nt,
                             preferred_element_type=jnp.float32)
        lg += lax.dot_general(jnp.ones((_N, 1), jnp.float32), bj_ref[0], nt,
                              preferred_element_type=jnp.float32)
        colid = lax.broadcasted_iota(jnp.int32, (_N, _BV), 1)
        lim = jnp.where(i == _NB - 1, _V - (_NB - 1) * _BV, _BV)
        logits_s[slot] = jnp.where(colid < lim, lg, _NEG)

    @pl.when(i > 0)
    def _consume():
        b = i - 1

        ms = [mv_s[j] for j in range(_BEAM)]
        js = [mi_s[j] for j in range(_BEAM)]
        macc = jnp.full((_N, 128), _NEG, jnp.float32)
        for t in range(_CH):
            tv = logits_s[1 - slot, :, 128 * t:128 * (t + 1)]
            ti = jnp.full((_N, 128), b * _BV + 128 * t, jnp.int32)
            macc = jnp.maximum(macc, tv)
            for j in range(_BEAM):
                hi = jnp.maximum(tv, ms[j])
                lo = jnp.minimum(tv, ms[j])
                cmp = tv > ms[j]
                hi_i = jnp.where(cmp, ti, js[j])
                lo_i = jnp.where(cmp, js[j], ti)
                ms[j], tv, js[j], ti = hi, lo, hi_i, lo_i
        for j in range(_BEAM):
            mv_s[j] = ms[j]
            mi_s[j] = js[j]

        bm = jnp.max(macc, axis=1, keepdims=True)
        m_old = m_s[...][:, 0:1]
        m_new = jnp.maximum(m_old, bm)
        seacc = jnp.zeros((_N, 128), jnp.float32)
        for t in range(_CH):
            seacc += jnp.exp(logits_s[1 - slot, :, 128 * t:128 * (t + 1)]
                             - m_new)
        sumexp = jnp.sum(seacc, axis=1, keepdims=True)
        s_new = s_s[...][:, 0:1] * jnp.exp(m_old - m_new) + sumexp
        m_s[...] = jnp.broadcast_to(m_new, (_N, 128))
        s_s[...] = jnp.broadcast_to(s_new, (_N, 128))

    @pl.when(i == _NB)
    def _final():
        lane3 = lax.broadcasted_iota(jnp.int32, (_BEAM, _N, 128), 2)
        rowi = lax.broadcasted_iota(jnp.int32, (_BEAM, _N, 128), 1)
        m = m_s[...][:, 0:1]
        s = s_s[...][:, 0:1]
        lse = m + jnp.log(s)
        hlp_b = hlp_ref[...]
        adj = mv_s[...] + (hlp_b - lse)[None]
        hlp3 = jnp.broadcast_to(hlp_b[None], (_BEAM, _N, 128))
        ci = mi_s[...] + lane3
        flat = rowi * _V + ci
        laneo = lax.broadcasted_iota(jnp.int32, (1, 128), 1)
        ov = jnp.full((1, 128), 0.0, jnp.float32)
        op = jnp.full((1, 128), 0.0, jnp.float32)
        oh = jnp.zeros((1, 128), jnp.int32)
        ot = jnp.zeros((1, 128), jnp.int32)
        for k in range(_BEAM):
            g = jnp.max(adj)
            hit = adj == g
            f = jnp.min(jnp.where(hit, flat, _IBIG))
            sel = hit & (flat == f)
            hlp_at = jnp.min(jnp.where(sel, hlp3, jnp.float32(_IBIG)))
            tok = jnp.min(jnp.where(sel, ci, _IBIG))
            hyp = jnp.min(jnp.where(sel, rowi, _IBIG))
            ov = jnp.where(laneo == k, g, ov)
            op = jnp.where(laneo == k, jnp.exp(g - hlp_at), op)
            oh = jnp.where(laneo == k, hyp, oh)
            ot = jnp.where(laneo == k, tok, ot)
            adj = jnp.where(sel, _NEG, adj)
        outv_ref[...] = ov
        outp_ref[...] = op
        outh_ref[...] = oh
        outt_ref[...] = ot


def kernel(decoder_input, encoder_out, hyps_log_prob, emb, conv_w, Wp, bp,
           Wj, bj):
    f32 = jnp.float32
    cw = conv_w.reshape(_G, 4, 4, _CTX)
    eye = jnp.eye(_G, dtype=f32)
    m0 = jnp.einsum('goi,gh->giho', cw[..., 0], eye).reshape(_D, _D)
    m1 = jnp.einsum('goi,gh->giho', cw[..., 1], eye).reshape(_D, _D)

    ids = decoder_input.T.reshape(-1)
    e01 = _sc_gather(emb, ids)

    hlp_b = jnp.broadcast_to(hyps_log_prob, (_N, 128))
    bp2 = bp.reshape(1, _D)
    bj2 = jnp.pad(bj, (0, _NB * _BV - _V)).reshape(_NB, _BV, 1)

    last = _NB - 1
    outs = pl.pallas_call(
        _body,
        grid=(_NB + 1,),
        in_specs=[
            pl.BlockSpec((2 * _N, _D), lambda i: (0, 0)),
            pl.BlockSpec((_N, _D), lambda i: (0, 0)),
            pl.BlockSpec((_N, 128), lambda i: (0, 0)),
            pl.BlockSpec((_D, _D), lambda i: (0, 0)),
            pl.BlockSpec((_D, _D), lambda i: (0, 0)),
            pl.BlockSpec((_D, _D), lambda i: (0, 0)),
            pl.BlockSpec((1, _D), lambda i: (0, 0)),
            pl.BlockSpec((_BV, _D), lambda i: (jnp.minimum(i, last), 0)),
            pl.BlockSpec((1, _BV, 1), lambda i: (jnp.minimum(i, last), 0, 0)),
        ],
        out_specs=[pl.BlockSpec((1, 128), lambda i: (0, 0))] * 4,
        out_shape=[
            jax.ShapeDtypeStruct((1, 128), f32),
            jax.ShapeDtypeStruct((1, 128), f32),
            jax.ShapeDtypeStruct((1, 128), jnp.int32),
            jax.ShapeDtypeStruct((1, 128), jnp.int32),
        ],
        scratch_shapes=[
            pltpu.VMEM((_N, _D), f32),
            pltpu.VMEM((_N, 128), f32),
            pltpu.VMEM((_N, 128), f32),
            pltpu.VMEM((_BEAM, _N, 128), f32),
            pltpu.VMEM((_BEAM, _N, 128), jnp.int32),
            pltpu.VMEM((2, _N, _BV), f32),
        ],
    )(e01, encoder_out, hlp_b, m0, m1, Wp, bp2, Wj, bj2)

    ov, op, oh, ot = outs
    return (ov[0, :_BEAM], op[0, :_BEAM], oh[0, :_BEAM], ot[0, :_BEAM])
